# baseline (speedup 1.0000x reference)
import jax
import jax.numpy as jnp
from jax.experimental import pallas as pl
from jax.experimental.pallas import tpu as pltpu

_BIG = 100000000.0


def _cost_kernel(logits_ref, pbq_ref, bt_ref, lab_ref, area_ref, out_ref,
                 bp_ref):
    bi = out_ref.shape[0]
    q = out_ref.shape[2]

    @pl.when(pl.program_id(0) == 0)
    def _():
        cxb = bt_ref[:, 0, :]
        cyb = bt_ref[:, 1, :]
        wb = bt_ref[:, 2, :]
        hb = bt_ref[:, 3, :]
        bp_ref[0] = cxb
        bp_ref[1] = cyb
        bp_ref[2] = wb
        bp_ref[3] = hb
        x0b, x1b = cxb - 0.5 * wb, cxb + 0.5 * wb
        y0b, y1b = cyb - 0.5 * hb, cyb + 0.5 * hb
        bp_ref[4] = x0b
        bp_ref[5] = x1b
        bp_ref[6] = y0b
        bp_ref[7] = y1b
        bp_ref[8] = (x1b - x0b) * (y1b - y0b)

    li = logits_ref[...]
    m = jnp.max(li, axis=-1, keepdims=True)
    e = jnp.exp(li - m)
    p = e / jnp.sum(e, axis=-1, keepdims=True)

    labels = lab_ref[...]
    idx = jnp.broadcast_to(labels[None], (bi,) + labels.shape)
    g = jnp.take_along_axis(p, idx, axis=2)

    pb = pbq_ref[...]
    cxp, cyp = pb[:, :, 0:1], pb[:, :, 1:2]
    wp, hp = pb[:, :, 2:3], pb[:, :, 3:4]

    cxb, cyb = bp_ref[0:1], bp_ref[1:2]
    wb, hb = bp_ref[2:3], bp_ref[3:4]
    x0b, x1b = bp_ref[4:5], bp_ref[5:6]
    y0b, y1b = bp_ref[6:7], bp_ref[7:8]
    a2 = bp_ref[8:9]

    cost_bbox = 0.25 * (jnp.abs(cxp - cxb) + jnp.abs(cyp - cyb)
                        + jnp.abs(wp - wb) + jnp.abs(hp - hb))

    x0p, x1p = cxp - 0.5 * wp, cxp + 0.5 * wp
    y0p, y1p = cyp - 0.5 * hp, cyp + 0.5 * hp
    a1 = (x1p - x0p) * (y1p - y0p)

    wx = jnp.maximum(jnp.minimum(x1p, x1b) - jnp.maximum(x0p, x0b), 0.0)
    wy = jnp.maximum(jnp.minimum(y1p, y1b) - jnp.maximum(y0p, y0b), 0.0)
    inter = wx * wy
    union = (a1 + a2) - inter
    iou = inter / union

    wex = jnp.maximum(x1p, x1b) - jnp.minimum(x0p, x0b)
    wey = jnp.maximum(y1p, y1b) - jnp.minimum(y0p, y0b)
    enc = wex * wey
    cost = cost_bbox - g - iou + (enc - union) / enc

    mask = area_ref[...][None] > 0.0
    out_ref[...] = jnp.where(mask, cost, _BIG)


def kernel(pred_logits, pred_boxes, boxes, area, labels):
    b, q, c = pred_logits.shape
    bi = 8
    n_i = pl.cdiv(q, bi)

    out = pl.pallas_call(
        _cost_kernel,
        grid=(n_i,),
        in_specs=[
            pl.BlockSpec((bi, b, c), lambda ii: (ii, 0, 0)),
            pl.BlockSpec((bi, b, 4), lambda ii: (ii, 0, 0)),
            pl.BlockSpec((b, 4, q), lambda ii: (0, 0, 0)),
            pl.BlockSpec((b, q), lambda ii: (0, 0)),
            pl.BlockSpec((b, q), lambda ii: (0, 0)),
        ],
        out_specs=pl.BlockSpec((bi, b, q), lambda ii: (ii, 0, 0)),
        out_shape=jax.ShapeDtypeStruct((q, b, q), jnp.float32),
        scratch_shapes=[
            pltpu.VMEM((9, b, q), jnp.float32),
        ],
        compiler_params=pltpu.CompilerParams(
            dimension_semantics=("arbitrary",),
        ),
        name="hungarian_cost",
    )(
        pred_logits.transpose(1, 0, 2),
        pred_boxes.transpose(1, 0, 2),
        boxes.transpose(0, 2, 1),
        labels.astype(jnp.int32),
        area,
    )
    return out.transpose(1, 0, 2)


# speedup vs baseline: 1.0747x; 1.0747x over previous
import jax
import jax.numpy as jnp
from jax.experimental import pallas as pl
from jax.experimental.pallas import tpu as pltpu

_BIG = 100000000.0


def _cost_kernel(logits_ref, pbq_ref, bt_ref, lab_ref, area_ref, out_ref,
                 bp_ref):
    bi = out_ref.shape[0]
    q = out_ref.shape[2]

    @pl.when(pl.program_id(0) == 0)
    def _():
        cxb = bt_ref[:, 0, :]
        cyb = bt_ref[:, 1, :]
        wb = bt_ref[:, 2, :]
        hb = bt_ref[:, 3, :]
        bp_ref[0] = cxb
        bp_ref[1] = cyb
        bp_ref[2] = 0.5 * wb
        bp_ref[3] = 0.5 * hb
        bp_ref[4] = wb
        bp_ref[5] = hb
        x0b, x1b = cxb - 0.5 * wb, cxb + 0.5 * wb
        y0b, y1b = cyb - 0.5 * hb, cyb + 0.5 * hb
        bp_ref[6] = (x1b - x0b) * (y1b - y0b)

    li = logits_ref[...]
    m = jnp.max(li, axis=-1, keepdims=True)
    e = jnp.exp(li - m)
    p = e / jnp.sum(e, axis=-1, keepdims=True)

    labels = lab_ref[...]
    idx = jnp.broadcast_to(labels[None], (bi,) + labels.shape)
    g = jnp.take_along_axis(p, idx, axis=2)

    pb = pbq_ref[...]
    cxp, cyp = pb[:, :, 0:1], pb[:, :, 1:2]
    wp, hp = pb[:, :, 2:3], pb[:, :, 3:4]

    cxb, cyb = bp_ref[0:1], bp_ref[1:2]
    hwb, hhb = bp_ref[2:3], bp_ref[3:4]
    wb, hb = bp_ref[4:5], bp_ref[5:6]
    a2 = bp_ref[6:7]

    adx = jnp.abs(cxp - cxb)
    ady = jnp.abs(cyp - cyb)
    adw = jnp.abs(wp - wb)
    adh = jnp.abs(hp - hb)
    cost_bbox = 0.25 * (adx + ady + adw + adh)

    hwp, hhp = 0.5 * wp, 0.5 * hp
    a1 = wp * hp
    sx = hwp + hwb
    sy = hhp + hhb
    tx = jnp.maximum(adx, 0.5 * adw)
    ty = jnp.maximum(ady, 0.5 * adh)

    wx = jnp.maximum(sx - tx, 0.0)
    wy = jnp.maximum(sy - ty, 0.0)
    inter = wx * wy
    union = (a1 + a2) - inter
    iou = inter / union

    enc = (sx + tx) * (sy + ty)
    cost = cost_bbox - g - iou + (enc - union) / enc

    mask = area_ref[...][None] > 0.0
    out_ref[...] = jnp.where(mask, cost, _BIG)


def kernel(pred_logits, pred_boxes, boxes, area, labels):
    b, q, c = pred_logits.shape
    bi = 8
    n_i = pl.cdiv(q, bi)

    out = pl.pallas_call(
        _cost_kernel,
        grid=(n_i,),
        in_specs=[
            pl.BlockSpec((bi, b, c), lambda ii: (ii, 0, 0)),
            pl.BlockSpec((bi, b, 4), lambda ii: (ii, 0, 0)),
            pl.BlockSpec((b, 4, q), lambda ii: (0, 0, 0)),
            pl.BlockSpec((b, q), lambda ii: (0, 0)),
            pl.BlockSpec((b, q), lambda ii: (0, 0)),
        ],
        out_specs=pl.BlockSpec((bi, b, q), lambda ii: (ii, 0, 0)),
        out_shape=jax.ShapeDtypeStruct((q, b, q), jnp.float32),
        scratch_shapes=[
            pltpu.VMEM((7, b, q), jnp.float32),
        ],
        compiler_params=pltpu.CompilerParams(
            dimension_semantics=("arbitrary",),
        ),
        name="hungarian_cost",
    )(
        pred_logits.transpose(1, 0, 2),
        pred_boxes.transpose(1, 0, 2),
        boxes.transpose(0, 2, 1),
        labels.astype(jnp.int32),
        area,
    )
    return out.transpose(1, 0, 2)


# speedup vs baseline: 1.0994x; 1.0230x over previous
import jax
import jax.numpy as jnp
from jax.experimental import pallas as pl
from jax.experimental.pallas import tpu as pltpu

_BIG = 100000000.0


def _cost_kernel(logits_ref, pbq_ref, bt_ref, lab_ref, area_ref, out_ref,
                 bp_ref):
    bi = out_ref.shape[0]
    q = out_ref.shape[2]

    @pl.when(pl.program_id(0) == 0)
    def _():
        cxb = bt_ref[:, 0, :]
        cyb = bt_ref[:, 1, :]
        wb = bt_ref[:, 2, :]
        hb = bt_ref[:, 3, :]
        bp_ref[0] = cxb
        bp_ref[1] = cyb
        bp_ref[2] = 0.5 * wb
        bp_ref[3] = 0.5 * hb
        bp_ref[4] = wb
        bp_ref[5] = hb
        x0b, x1b = cxb - 0.5 * wb, cxb + 0.5 * wb
        y0b, y1b = cyb - 0.5 * hb, cyb + 0.5 * hb
        bp_ref[6] = (x1b - x0b) * (y1b - y0b)

    li = logits_ref[...]
    m = jnp.max(li, axis=-1, keepdims=True)
    e = jnp.exp(li - m)
    p = e * (1.0 / jnp.sum(e, axis=-1, keepdims=True))

    labels = lab_ref[...]
    idx = jnp.broadcast_to(labels[None], (bi,) + labels.shape)
    g = jnp.take_along_axis(p, idx, axis=2)

    pb = pbq_ref[...]
    cxp, cyp = pb[:, :, 0:1], pb[:, :, 1:2]
    wp, hp = pb[:, :, 2:3], pb[:, :, 3:4]

    cxb, cyb = bp_ref[0:1], bp_ref[1:2]
    hwb, hhb = bp_ref[2:3], bp_ref[3:4]
    wb, hb = bp_ref[4:5], bp_ref[5:6]
    a2 = bp_ref[6:7]

    adx = jnp.abs(cxp - cxb)
    ady = jnp.abs(cyp - cyb)
    adw = jnp.abs(wp - wb)
    adh = jnp.abs(hp - hb)
    cost_bbox = 0.25 * (adx + ady + adw + adh)

    hwp, hhp = 0.5 * wp, 0.5 * hp
    a1 = wp * hp
    sx = hwp + hwb
    sy = hhp + hhb
    tx = jnp.maximum(adx, 0.5 * adw)
    ty = jnp.maximum(ady, 0.5 * adh)

    wx = jnp.maximum(sx - tx, 0.0)
    wy = jnp.maximum(sy - ty, 0.0)
    inter = wx * wy
    union = (a1 + a2) - inter
    enc = (sx + tx) * (sy + ty)
    cost = (cost_bbox - g + 1.0
            - (inter * enc + union * union) / (union * enc))

    mask = area_ref[...][None] > 0.0
    out_ref[...] = jnp.where(mask, cost, _BIG)


def kernel(pred_logits, pred_boxes, boxes, area, labels):
    b, q, c = pred_logits.shape
    bi = 16
    n_i = pl.cdiv(q, bi)

    out = pl.pallas_call(
        _cost_kernel,
        grid=(n_i,),
        in_specs=[
            pl.BlockSpec((bi, b, c), lambda ii: (ii, 0, 0)),
            pl.BlockSpec((bi, b, 4), lambda ii: (ii, 0, 0)),
            pl.BlockSpec((b, 4, q), lambda ii: (0, 0, 0)),
            pl.BlockSpec((b, q), lambda ii: (0, 0)),
            pl.BlockSpec((b, q), lambda ii: (0, 0)),
        ],
        out_specs=pl.BlockSpec((bi, b, q), lambda ii: (ii, 0, 0)),
        out_shape=jax.ShapeDtypeStruct((q, b, q), jnp.float32),
        scratch_shapes=[
            pltpu.VMEM((7, b, q), jnp.float32),
        ],
        compiler_params=pltpu.CompilerParams(
            dimension_semantics=("arbitrary",),
        ),
        name="hungarian_cost",
    )(
        pred_logits.transpose(1, 0, 2),
        pred_boxes.transpose(1, 0, 2),
        boxes.transpose(0, 2, 1),
        labels.astype(jnp.int32),
        area,
    )
    return out.transpose(1, 0, 2)


# speedup vs baseline: 1.1458x; 1.0422x over previous
import jax
import jax.numpy as jnp
from jax.experimental import pallas as pl
from jax.experimental.pallas import tpu as pltpu

_BIG = 100000000.0


def _cost_kernel(logits_ref, pbq_ref, bt_ref, lab_ref, area_ref, out_ref,
                 bp_ref):
    bi = out_ref.shape[0]
    q = out_ref.shape[2]

    @pl.when(pl.program_id(0) == 0)
    def _():
        cxb = bt_ref[:, 0, :]
        cyb = bt_ref[:, 1, :]
        wb = bt_ref[:, 2, :]
        hb = bt_ref[:, 3, :]
        bp_ref[0] = cxb
        bp_ref[1] = cyb
        bp_ref[2] = 0.5 * wb
        bp_ref[3] = 0.5 * hb
        bp_ref[4] = wb
        bp_ref[5] = hb
        x0b, x1b = cxb - 0.5 * wb, cxb + 0.5 * wb
        y0b, y1b = cyb - 0.5 * hb, cyb + 0.5 * hb
        bp_ref[6] = (x1b - x0b) * (y1b - y0b)

    li = logits_ref[...]
    m = jnp.max(li, axis=-1, keepdims=True)
    e = jnp.exp(li - m)
    p = e * (1.0 / jnp.sum(e, axis=-1, keepdims=True))

    labels = lab_ref[...]
    idx = jnp.broadcast_to(labels[None], (bi,) + labels.shape)
    g = jnp.take_along_axis(p, idx, axis=2)

    pb = pbq_ref[...]
    cxp, cyp = pb[:, :, 0:1], pb[:, :, 1:2]
    wp, hp = pb[:, :, 2:3], pb[:, :, 3:4]

    cxb, cyb = bp_ref[0:1], bp_ref[1:2]
    hwb, hhb = bp_ref[2:3], bp_ref[3:4]
    wb, hb = bp_ref[4:5], bp_ref[5:6]
    a2 = bp_ref[6:7]

    adx = jnp.abs(cxp - cxb)
    ady = jnp.abs(cyp - cyb)
    adw = jnp.abs(wp - wb)
    adh = jnp.abs(hp - hb)
    cost_bbox = 0.25 * (adx + ady + adw + adh)

    hwp, hhp = 0.5 * wp, 0.5 * hp
    a1 = wp * hp
    sx = hwp + hwb
    sy = hhp + hhb
    tx = jnp.maximum(adx, 0.5 * adw)
    ty = jnp.maximum(ady, 0.5 * adh)

    wx = jnp.maximum(sx - tx, 0.0)
    wy = jnp.maximum(sy - ty, 0.0)
    inter = wx * wy
    union = (a1 + a2) - inter
    enc = (sx + tx) * (sy + ty)
    cost = (cost_bbox - g + 1.0
            - (inter * enc + union * union) / (union * enc))

    mask = area_ref[...][None] > 0.0
    out_ref[...] = jnp.where(mask, cost, _BIG)


def kernel(pred_logits, pred_boxes, boxes, area, labels):
    b, q, c = pred_logits.shape
    bi = 36
    n_i = pl.cdiv(q, bi)

    out = pl.pallas_call(
        _cost_kernel,
        grid=(n_i,),
        in_specs=[
            pl.BlockSpec((bi, b, c), lambda ii: (ii, 0, 0)),
            pl.BlockSpec((bi, b, 4), lambda ii: (ii, 0, 0)),
            pl.BlockSpec((b, 4, q), lambda ii: (0, 0, 0)),
            pl.BlockSpec((b, q), lambda ii: (0, 0)),
            pl.BlockSpec((b, q), lambda ii: (0, 0)),
        ],
        out_specs=pl.BlockSpec((bi, b, q), lambda ii: (ii, 0, 0)),
        out_shape=jax.ShapeDtypeStruct((q, b, q), jnp.float32),
        scratch_shapes=[
            pltpu.VMEM((7, b, q), jnp.float32),
        ],
        compiler_params=pltpu.CompilerParams(
            dimension_semantics=("arbitrary",),
        ),
        name="hungarian_cost",
    )(
        pred_logits.transpose(1, 0, 2),
        pred_boxes.transpose(1, 0, 2),
        boxes.transpose(0, 2, 1),
        labels.astype(jnp.int32),
        area,
    )
    return out.transpose(1, 0, 2)


# speedup vs baseline: 1.1503x; 1.0039x over previous
import jax
import jax.numpy as jnp
from jax.experimental import pallas as pl
from jax.experimental.pallas import tpu as pltpu

_BIG = 100000000.0


def _cost_kernel(logits_ref, pbq_ref, bt_ref, lab_ref, area_ref, out_ref,
                 bp_ref):
    bi = out_ref.shape[0]
    q = out_ref.shape[2]

    @pl.when(pl.program_id(0) == 0)
    def _():
        cxb = bt_ref[:, 0, :]
        cyb = bt_ref[:, 1, :]
        wb = bt_ref[:, 2, :]
        hb = bt_ref[:, 3, :]
        bp_ref[0] = cxb
        bp_ref[1] = cyb
        bp_ref[2] = 0.5 * wb
        bp_ref[3] = 0.5 * hb
        bp_ref[4] = wb
        bp_ref[5] = hb
        x0b, x1b = cxb - 0.5 * wb, cxb + 0.5 * wb
        y0b, y1b = cyb - 0.5 * hb, cyb + 0.5 * hb
        bp_ref[6] = (x1b - x0b) * (y1b - y0b)

    li = logits_ref[...]
    m = jnp.max(li, axis=-1, keepdims=True)
    e = jnp.exp(li - m)
    p = e * (1.0 / jnp.sum(e, axis=-1, keepdims=True))

    labels = lab_ref[...]
    idx = jnp.broadcast_to(labels[None], (bi,) + labels.shape)
    g = jnp.take_along_axis(p, idx, axis=2)

    pb = pbq_ref[...]
    cxp, cyp = pb[:, :, 0:1], pb[:, :, 1:2]
    wp, hp = pb[:, :, 2:3], pb[:, :, 3:4]

    cxb, cyb = bp_ref[0:1], bp_ref[1:2]
    hwb, hhb = bp_ref[2:3], bp_ref[3:4]
    wb, hb = bp_ref[4:5], bp_ref[5:6]
    a2 = bp_ref[6:7]

    adx = jnp.abs(cxp - cxb)
    ady = jnp.abs(cyp - cyb)
    adw = jnp.abs(wp - wb)
    adh = jnp.abs(hp - hb)
    cost_bbox = 0.25 * (adx + ady + adw + adh)

    hwp, hhp = 0.5 * wp, 0.5 * hp
    a1 = wp * hp
    sx = hwp + hwb
    sy = hhp + hhb
    tx = jnp.maximum(adx, 0.5 * adw)
    ty = jnp.maximum(ady, 0.5 * adh)

    wx = jnp.maximum(sx - tx, 0.0)
    wy = jnp.maximum(sy - ty, 0.0)
    inter = wx * wy
    union = (a1 + a2) - inter
    enc = (sx + tx) * (sy + ty)
    cost = (cost_bbox - g + 1.0
            - (inter * enc + union * union) / (union * enc))

    mask = area_ref[...][None] > 0.0
    out_ref[...] = jnp.where(mask, cost, _BIG)


def kernel(pred_logits, pred_boxes, boxes, area, labels):
    b, q, c = pred_logits.shape
    bi = 45
    n_i = pl.cdiv(q, bi)

    out = pl.pallas_call(
        _cost_kernel,
        grid=(n_i,),
        in_specs=[
            pl.BlockSpec((bi, b, c), lambda ii: (ii, 0, 0)),
            pl.BlockSpec((bi, b, 4), lambda ii: (ii, 0, 0)),
            pl.BlockSpec((b, 4, q), lambda ii: (0, 0, 0)),
            pl.BlockSpec((b, q), lambda ii: (0, 0)),
            pl.BlockSpec((b, q), lambda ii: (0, 0)),
        ],
        out_specs=pl.BlockSpec((bi, b, q), lambda ii: (ii, 0, 0)),
        out_shape=jax.ShapeDtypeStruct((q, b, q), jnp.float32),
        scratch_shapes=[
            pltpu.VMEM((7, b, q), jnp.float32),
        ],
        compiler_params=pltpu.CompilerParams(
            dimension_semantics=("arbitrary",),
        ),
        name="hungarian_cost",
    )(
        pred_logits.transpose(1, 0, 2),
        pred_boxes.transpose(1, 0, 2),
        boxes.transpose(0, 2, 1),
        labels.astype(jnp.int32),
        area,
    )
    return out.transpose(1, 0, 2)
